# Initial kernel scaffold; baseline (speedup 1.0000x reference)
import jax
import jax.numpy as jnp
from jax.experimental import pallas as pl


def kernel(xyz, voxels, W_conv, b_conv, ln_gamma, ln_beta):
    raise NotImplementedError("write your pallas kernel here")


# speedup vs baseline: 5.5766x; 5.5766x over previous
import functools

import jax
import jax.numpy as jnp
import numpy as np
from jax import lax
from jax.experimental import pallas as pl
from jax.experimental.pallas import tpu as pltpu

B, N, C, OUT = 4, 4096, 32, 64
S = 512
K = 32
R2 = np.float32(0.2 ** 2)
NB = N // 128
SC = 64
_NEG = np.float32(-1e30)


def _fps_body(xp_ref, nx_ref):
    xs = xp_ref[0]
    ys = xp_ref[1]
    zs = xp_ref[2]
    flat_iota = (lax.broadcasted_iota(jnp.int32, (1, 32, 128), 1) * 128
                 + lax.broadcasted_iota(jnp.int32, (1, 32, 128), 2))

    cx0 = xp_ref[0, :, 0:1, 0:1]
    cy0 = xp_ref[1, :, 0:1, 0:1]
    cz0 = xp_ref[2, :, 0:1, 0:1]
    dist0 = jnp.full((B, 32, 128), 1e10, dtype=jnp.float32)

    def step(i, carry):
        dist, cx, cy, cz = carry
        nx_ref[:, pl.ds(i, 1), :] = jnp.concatenate(
            [cx, cy, cz], axis=-1).reshape(B, 1, 3)
        dx = xs - cx
        dy = ys - cy
        dz = zs - cz
        d = dx * dx + dy * dy + dz * dz
        dist = jnp.minimum(dist, d)
        mx = jnp.max(dist, axis=(1, 2), keepdims=True)
        cand = jnp.where(dist == mx, flat_iota, jnp.int32(N))
        far = jnp.min(cand, axis=(1, 2), keepdims=True)
        sel = (flat_iota == far).astype(jnp.float32)
        ncx = jnp.sum(xs * sel, axis=(1, 2), keepdims=True)
        ncy = jnp.sum(ys * sel, axis=(1, 2), keepdims=True)
        ncz = jnp.sum(zs * sel, axis=(1, 2), keepdims=True)
        return dist, ncx, ncy, ncz

    lax.fori_loop(0, S, step, (dist0, cx0, cy0, cz0))


def _run_fps(xyz):
    xp = xyz.transpose(2, 0, 1).reshape(3, B, 32, 128)
    return pl.pallas_call(
        _fps_body,
        out_shape=jax.ShapeDtypeStruct((B, S, 3), jnp.float32),
    )(xp)


def _group_body(xyzt_ref, xyz_ref, vox_ref, nxyz_ref, wxyz_ref, wpts_ref,
                bias_ref, gamma_ref, beta_ref, out_ref, p_scr, m_scr):
    p = (jnp.dot(vox_ref[0], wpts_ref[...], preferred_element_type=jnp.float32)
         + jnp.dot(xyz_ref[0], wxyz_ref[...], preferred_element_type=jnp.float32)
         + bias_ref[...])
    p_scr[...] = p

    xs = xyzt_ref[0, 0:1, :]
    ys = xyzt_ref[0, 1:2, :]
    zs = xyzt_ref[0, 2:3, :]

    n_iota = lax.broadcasted_iota(jnp.int32, (128, 128), 0)
    m_iota = lax.broadcasted_iota(jnp.int32, (128, 128), 1)
    u128 = (n_iota <= m_iota).astype(jnp.float32)
    b_i = lax.broadcasted_iota(jnp.int32, (K, K), 0)
    b_j = lax.broadcasted_iota(jnp.int32, (K, K), 1)
    l32 = (b_i < b_j).astype(jnp.float32)

    gamma = gamma_ref[...]
    beta = beta_ref[...]
    wxyz = wxyz_ref[...]

    def chunk(sc, _):
        nx = nxyz_ref[0, pl.ds(sc * SC, SC), :]
        cx = nx[:, 0:1]
        cy = nx[:, 1:2]
        cz = nx[:, 2:3]
        dx = cx - xs
        dy = cy - ys
        dz = cz - zs
        d = dx * dx + dy * dy + dz * dz
        maskf = (d <= R2).astype(jnp.float32)
        cw = jnp.dot(maskf.reshape(SC * NB, 128), u128,
                     preferred_element_type=jnp.float32).reshape(SC, NB, 128)
        bs = cw[:, :, 127]
        offs = jnp.dot(bs, (lax.broadcasted_iota(jnp.int32, (NB, NB), 0)
                            < lax.broadcasted_iota(jnp.int32, (NB, NB), 1)
                            ).astype(jnp.float32),
                       preferred_element_type=jnp.float32)
        c = cw + offs[:, :, None]
        m2 = (maskf.reshape(SC, NB, 128) > 0.0) & (c <= jnp.float32(K))
        m_scr[...] = jnp.where(m2, 0.0, _NEG).reshape(SC, N)
        nb_stop = jnp.max(jnp.sum((offs < jnp.float32(K)).astype(jnp.float32),
                                  axis=1)).astype(jnp.int32)

        def blk(nb, acc):
            pb = p_scr[pl.ds(nb * 128, 128), :]
            mb = m_scr[:, pl.ds(nb * 128, 128)]
            t = mb[:, :, None] + pb[None, :, :]
            return jnp.maximum(acc, jnp.max(t, axis=1))

        acc = lax.fori_loop(0, nb_stop, blk,
                            jnp.full((SC, OUT), _NEG, dtype=jnp.float32))
        x = acc - jnp.dot(nx, wxyz, preferred_element_type=jnp.float32)
        mean = jnp.mean(x, axis=-1, keepdims=True)
        xc = x - mean
        var = jnp.mean(xc * xc, axis=-1, keepdims=True)
        y = xc * lax.rsqrt(var + jnp.float32(1e-5)) * gamma + beta
        out_ref[0, pl.ds(sc * SC, SC), :] = y
        return 0

    lax.fori_loop(0, S // SC, chunk, 0)


def _run_group(xyz, voxels, new_xyz, w_xyz, w_pts, bias, gamma, beta):
    xyzt = xyz.transpose(0, 2, 1)
    grid = (B,)
    return pl.pallas_call(
        _group_body,
        grid=grid,
        in_specs=[
            pl.BlockSpec((1, 3, N), lambda b: (b, 0, 0)),
            pl.BlockSpec((1, N, 3), lambda b: (b, 0, 0)),
            pl.BlockSpec((1, N, C), lambda b: (b, 0, 0)),
            pl.BlockSpec((1, S, 3), lambda b: (b, 0, 0)),
            pl.BlockSpec((3, OUT), lambda b: (0, 0)),
            pl.BlockSpec((C, OUT), lambda b: (0, 0)),
            pl.BlockSpec((1, OUT), lambda b: (0, 0)),
            pl.BlockSpec((1, OUT), lambda b: (0, 0)),
            pl.BlockSpec((1, OUT), lambda b: (0, 0)),
        ],
        out_specs=pl.BlockSpec((1, S, OUT), lambda b: (b, 0, 0)),
        out_shape=jax.ShapeDtypeStruct((B, S, OUT), jnp.float32),
        scratch_shapes=[
            pltpu.VMEM((N, OUT), jnp.float32),
            pltpu.VMEM((SC, N), jnp.float32),
        ],
    )(xyzt, xyz, voxels, new_xyz, w_xyz, w_pts, bias, gamma, beta)


def kernel(xyz, voxels, W_conv, b_conv, ln_gamma, ln_beta):
    new_xyz = _run_fps(xyz)
    w_t = W_conv.T
    x = _run_group(xyz, voxels, new_xyz,
                   w_t[:3], w_t[3:],
                   b_conv.reshape(1, OUT),
                   ln_gamma.reshape(1, OUT),
                   ln_beta.reshape(1, OUT))
    return (x, new_xyz)
